# baseline (speedup 1.0000x reference)
import functools

import jax
import jax.numpy as jnp
from jax import lax
from jax.experimental import pallas as pl
from jax.experimental.pallas import tpu as pltpu
from jax.experimental.pallas import tpu_sc as plsc

VOCAB = 1000000
EMBED_DIM = 32
BATCH = 16384

NUM_CORES = 2
NUM_SUBCORES = 16
NUM_WORKERS = NUM_CORES * NUM_SUBCORES
ROWS_PER_WORKER = BATCH // NUM_WORKERS
LANES = 128
NSLOTS = 16
NROUNDS = ROWS_PER_WORKER // NSLOTS


def _gather_body(tabT_hbm, idx_hbm, outT_hbm, idx_sh, idx_s, colT, *rest):
    slots = rest[:NSLOTS]
    sems = rest[NSLOTS:]
    sid = lax.axis_index("s")
    wid = sid * NUM_CORES + lax.axis_index("c")
    base = wid * ROWS_PER_WORKER
    pltpu.sync_copy(idx_hbm.at[pl.ds(base, ROWS_PER_WORKER)], idx_sh.at[sid])
    pltpu.sync_copy(idx_sh.at[sid], idx_s)

    lane16 = lax.iota(jnp.int32, 16)

    def fire(u, j):
        tile_base = pl.multiple_of((idx_s[j] >> 7) * LANES, LANES)
        pltpu.async_copy(
            tabT_hbm.at[:, pl.ds(tile_base, LANES)], slots[u], sems[u]
        )

    def land(u, j):
        pltpu.make_async_copy(
            tabT_hbm.at[:, pl.ds(0, LANES)], slots[u], sems[u]
        ).wait()
        r = jnp.full((16,), idx_s[j] & (LANES - 1), jnp.int32)
        jv = jnp.full((16,), j, jnp.int32)
        lo = plsc.load_gather(slots[u], [lane16, r])
        hi = plsc.load_gather(slots[u], [lane16 + 16, r])
        plsc.store_scatter(colT, [lane16, jv], lo)
        plsc.store_scatter(colT, [lane16 + 16, jv], hi)

    for u in range(NSLOTS):
        fire(u, u)

    def round_body(i, carry):
        for u in range(NSLOTS):
            land(u, (i - 1) * NSLOTS + u)

            @pl.when(i < NROUNDS)
            def _():
                fire(u, i * NSLOTS + u)

        return carry

    lax.fori_loop(1, NROUNDS + 1, round_body, 0, unroll=False)
    pltpu.sync_copy(colT, outT_hbm.at[:, pl.ds(base, ROWS_PER_WORKER)])


@jax.jit
def kernel(indices, table):
    idx = indices.astype(jnp.int32)
    tabT = table.T
    mesh = plsc.VectorSubcoreMesh(core_axis_name="c", subcore_axis_name="s")
    scratch = [
        pltpu.VMEM_SHARED((NUM_SUBCORES, ROWS_PER_WORKER), jnp.int32),
        pltpu.SMEM((ROWS_PER_WORKER,), jnp.int32),
        pltpu.VMEM((EMBED_DIM, ROWS_PER_WORKER), jnp.float32),
    ]
    scratch += [
        pltpu.VMEM((EMBED_DIM, LANES), jnp.float32) for _ in range(NSLOTS)
    ]
    scratch += [pltpu.SemaphoreType.DMA for _ in range(NSLOTS)]
    run = functools.partial(
        pl.kernel,
        mesh=mesh,
        out_type=jax.ShapeDtypeStruct((EMBED_DIM, BATCH), jnp.float32),
        scratch_types=scratch,
        compiler_params=pltpu.CompilerParams(needs_layout_passes=False),
    )(_gather_body)
    outT = run(tabT, idx)
    return outT.T


# speedup vs baseline: 1.0369x; 1.0369x over previous
import functools

import jax
import jax.numpy as jnp
from jax import lax
from jax.experimental import pallas as pl
from jax.experimental.pallas import tpu as pltpu
from jax.experimental.pallas import tpu_sc as plsc

VOCAB = 1000000
EMBED_DIM = 32
BATCH = 16384

NUM_CORES = 2
NUM_SUBCORES = 16
NUM_WORKERS = NUM_CORES * NUM_SUBCORES
ROWS_PER_WORKER = BATCH // NUM_WORKERS
LANES = 128
NSLOTS = 8
NROUNDS = ROWS_PER_WORKER // NSLOTS


def _gather_body(tabT_hbm, idx_hbm, outT_hbm, idx_sh, idx_s, colT, *rest):
    slots = rest[:NSLOTS]
    sems = rest[NSLOTS:]
    sid = lax.axis_index("s")
    wid = sid * NUM_CORES + lax.axis_index("c")
    base = wid * ROWS_PER_WORKER
    pltpu.sync_copy(idx_hbm.at[pl.ds(base, ROWS_PER_WORKER)], idx_sh.at[sid])
    pltpu.sync_copy(idx_sh.at[sid], idx_s)

    lane16 = lax.iota(jnp.int32, 16)

    def fire(u, j):
        tile_base = pl.multiple_of((idx_s[j] >> 7) * LANES, LANES)
        pltpu.async_copy(
            tabT_hbm.at[:, pl.ds(tile_base, LANES)], slots[u], sems[u]
        )

    def land(u, j):
        pltpu.make_async_copy(
            tabT_hbm.at[:, pl.ds(0, LANES)], slots[u], sems[u]
        ).wait()
        r = jnp.full((16,), idx_s[j] & (LANES - 1), jnp.int32)
        jv = jnp.full((16,), j, jnp.int32)
        lo = plsc.load_gather(slots[u], [lane16, r])
        hi = plsc.load_gather(slots[u], [lane16 + 16, r])
        plsc.store_scatter(colT, [lane16, jv], lo)
        plsc.store_scatter(colT, [lane16 + 16, jv], hi)

    for u in range(NSLOTS):
        fire(u, u)

    def round_body(i, carry):
        for u in range(NSLOTS):
            land(u, (i - 1) * NSLOTS + u)

            @pl.when(i < NROUNDS)
            def _():
                fire(u, i * NSLOTS + u)

        return carry

    lax.fori_loop(1, NROUNDS + 1, round_body, 0, unroll=False)
    pltpu.sync_copy(colT, outT_hbm.at[:, pl.ds(base, ROWS_PER_WORKER)])


@jax.jit
def kernel(indices, table):
    idx = indices.astype(jnp.int32)
    tabT = table.T
    mesh = plsc.VectorSubcoreMesh(core_axis_name="c", subcore_axis_name="s")
    scratch = [
        pltpu.VMEM_SHARED((NUM_SUBCORES, ROWS_PER_WORKER), jnp.int32),
        pltpu.SMEM((ROWS_PER_WORKER,), jnp.int32),
        pltpu.VMEM((EMBED_DIM, ROWS_PER_WORKER), jnp.float32),
    ]
    scratch += [
        pltpu.VMEM((EMBED_DIM, LANES), jnp.float32) for _ in range(NSLOTS)
    ]
    scratch += [pltpu.SemaphoreType.DMA for _ in range(NSLOTS)]
    run = functools.partial(
        pl.kernel,
        mesh=mesh,
        out_type=jax.ShapeDtypeStruct((EMBED_DIM, BATCH), jnp.float32),
        scratch_types=scratch,
        compiler_params=pltpu.CompilerParams(needs_layout_passes=False),
    )(_gather_body)
    outT = run(tabT, idx)
    return outT.T
